# Initial kernel scaffold; baseline (speedup 1.0000x reference)
import jax
import jax.numpy as jnp
from jax.experimental import pallas as pl


def kernel(pointcloud, params):
    raise NotImplementedError("write your pallas kernel here")


# speedup vs baseline: 6.7994x; 6.7994x over previous
import functools

import jax
import jax.numpy as jnp
import numpy as np
from jax import lax
from jax.experimental import pallas as pl
from jax.experimental.pallas import tpu as pltpu
from jax.experimental.pallas import tpu_sc as plsc

B, N = 8, 8192
NPOINTS = [512, 128, 32]
RADII = [0.04, 0.08, 0.16]
NSAMPLES = [64, 32, 16]
BN_EPS = 1e-5
MLP_DIMS = [[3, 32, 32, 64], [67, 128, 128, 256], [259, 256, 512, 512],
            [515, 512, 1024, 1024]]

_NW = 32


def _fps_stage(xs, ys, zs, npoint, cx_ref, cy_ref, cz_ref, dists_ref):
    Bb, Nn = xs.shape
    iota = lax.broadcasted_iota(jnp.int32, (Bb, Nn), 1)
    iota_s = lax.broadcasted_iota(jnp.int32, (Bb, npoint), 1)
    cx_ref[...] = jnp.zeros((Bb, npoint), jnp.float32)
    cy_ref[...] = jnp.zeros((Bb, npoint), jnp.float32)
    cz_ref[...] = jnp.zeros((Bb, npoint), jnp.float32)
    dists_ref[...] = jnp.full((Bb, Nn), 1e10, jnp.float32)

    def body(i, far):
        sel = iota == far
        cx = jnp.sum(jnp.where(sel, xs, 0.0), axis=1, keepdims=True)
        cy = jnp.sum(jnp.where(sel, ys, 0.0), axis=1, keepdims=True)
        cz = jnp.sum(jnp.where(sel, zs, 0.0), axis=1, keepdims=True)
        hot = iota_s == i
        cx_ref[...] = jnp.where(hot, cx, cx_ref[...])
        cy_ref[...] = jnp.where(hot, cy, cy_ref[...])
        cz_ref[...] = jnp.where(hot, cz, cz_ref[...])
        d = (xs - cx) ** 2 + (ys - cy) ** 2 + (zs - cz) ** 2
        dists = jnp.minimum(dists_ref[...], d)
        dists_ref[...] = dists
        m = jnp.max(dists, axis=1, keepdims=True)
        far = jnp.min(jnp.where(dists == m, iota, Nn), axis=1, keepdims=True)
        return far.astype(jnp.int32)

    lax.fori_loop(0, npoint, body, jnp.zeros((Bb, 1), jnp.int32))


def _fps_all_kernel(x_ref, y_ref, z_ref,
                    cx0, cy0, cz0, cx1, cy1, cz1, cx2, cy2, cz2,
                    d0, d1, d2):
    _fps_stage(x_ref[...], y_ref[...], z_ref[...], NPOINTS[0],
               cx0, cy0, cz0, d0)
    _fps_stage(cx0[...], cy0[...], cz0[...], NPOINTS[1], cx1, cy1, cz1, d1)
    _fps_stage(cx1[...], cy1[...], cz1[...], NPOINTS[2], cx2, cy2, cz2, d2)


def _fps_all(x, y, z):
    outs = [jax.ShapeDtypeStruct((B, s), jnp.float32)
            for s in NPOINTS for _ in range(3)]
    return pl.pallas_call(
        _fps_all_kernel,
        out_shape=tuple(outs),
        scratch_shapes=[pltpu.VMEM((B, N), jnp.float32),
                        pltpu.VMEM((B, NPOINTS[0]), jnp.float32),
                        pltpu.VMEM((B, NPOINTS[1]), jnp.float32)],
    )(x, y, z)


def _ballq_kernel(nxyz_ref, xt_ref, out_ref, *, radius, nsample):
    a = nxyz_ref[0]
    bt = xt_ref[0]
    S = a.shape[0]
    Nn = bt.shape[1]
    aa = jnp.sum(a[:, 0:3] * a[:, 0:3], axis=1, keepdims=True)
    bb = jnp.sum(bt[0:3, :] * bt[0:3, :], axis=0, keepdims=True)
    ab = jnp.dot(a, bt, preferred_element_type=jnp.float32)
    sqr = aa + bb - 2.0 * ab
    iota = lax.broadcasted_iota(jnp.int32, (S, Nn), 1)
    val0 = jnp.where(sqr > radius * radius, Nn, iota)
    iota_k = lax.broadcasted_iota(jnp.int32, (S, nsample), 1)

    def body(k, state):
        val, acc = state
        m = jnp.min(val, axis=1, keepdims=True)
        acc = jnp.where(iota_k == k, m, acc)
        val = jnp.where(val == m, Nn, val)
        return val, acc

    _, idx = lax.fori_loop(0, nsample, body,
                           (val0, jnp.zeros((S, nsample), jnp.int32)))
    first = idx[:, 0:1]
    first = jnp.where(first == Nn, 0, first)
    idx = jnp.where(idx == Nn, first, idx)
    out_ref[0] = idx


def _ball_query(new_xyz8, xt8, radius, nsample):
    _, S, _ = new_xyz8.shape
    Nn = xt8.shape[2]
    return pl.pallas_call(
        functools.partial(_ballq_kernel, radius=radius, nsample=nsample),
        grid=(B,),
        in_specs=[pl.BlockSpec((1, S, 8), lambda b: (b, 0, 0)),
                  pl.BlockSpec((1, 8, Nn), lambda b: (b, 0, 0))],
        out_specs=pl.BlockSpec((1, S, nsample), lambda b: (b, 0, 0)),
        out_shape=jax.ShapeDtypeStruct((B, S, nsample), jnp.int32),
    )(new_xyz8, xt8)


def _gather_rows(table, idx):
    rows, d = idx.shape[0], table.shape[1]
    rows_pw = rows // _NW
    nchunks = rows_pw // 128
    idx3 = idx.reshape(_NW, nchunks, 128)
    mesh = plsc.VectorSubcoreMesh(core_axis_name="c", subcore_axis_name="s")

    @functools.partial(
        pl.kernel, mesh=mesh,
        compiler_params=pltpu.CompilerParams(use_tc_tiling_on_sc=False),
        out_type=jax.ShapeDtypeStruct((rows, d), jnp.float32),
        scratch_types=[pltpu.VMEM((nchunks, 128), jnp.int32),
                       pltpu.VMEM((128, d), jnp.float32),
                       pltpu.SemaphoreType.DMA],
    )
    def k(table_hbm, idx_hbm, out_hbm, idx_v, rows_v, sem):
        wid = lax.axis_index("s") * 2 + lax.axis_index("c")
        pltpu.sync_copy(idx_hbm.at[wid], idx_v)

        def body(j, carry):
            pltpu.async_copy(table_hbm.at[idx_v.at[j]], rows_v, sem).wait()
            pltpu.sync_copy(
                rows_v, out_hbm.at[pl.ds(wid * rows_pw + j * 128, 128)])
            return carry

        lax.fori_loop(0, nchunks, body, 0)

    return k(table, idx3)


def _mlp_kernel(g_ref, c_ref, w1, s1, b1, w2, s2, b2, w3, s3, b3, out_ref,
                *, sb, ks):
    d = g_ref.shape[1]
    x = g_ref[...].reshape(sb, ks, d) - c_ref[0][:, None, :]
    x = x.reshape(sb * ks, d)
    for w, s, b in ((w1, s1, b1), (w2, s2, b2), (w3, s3, b3)):
        x = jnp.dot(x, w[...], preferred_element_type=jnp.float32)
        x = jnp.maximum(x * s[...] + b[...], 0.0)
    cout = x.shape[1]
    out_ref[0] = jnp.max(x.reshape(sb, ks, cout), axis=1)


def _mlp_max(g, cpad, weights, ks, sb):
    _, s_dim, d = cpad.shape
    (w1, s1, b1), (w2, s2, b2), (w3, s3, b3) = weights
    cout = w3.shape[1]
    n_sb = s_dim // sb
    wspec = [pl.BlockSpec(w.shape, lambda b, s: tuple([0] * w.ndim))
             for w in (w1, s1, b1, w2, s2, b2, w3, s3, b3)]
    return pl.pallas_call(
        functools.partial(_mlp_kernel, sb=sb, ks=ks),
        grid=(B, n_sb),
        in_specs=[pl.BlockSpec((sb * ks, d),
                               lambda b, s, n_sb=n_sb: (b * n_sb + s, 0)),
                  pl.BlockSpec((1, sb, d), lambda b, s: (b, s, 0))] + wspec,
        out_specs=pl.BlockSpec((1, sb, cout), lambda b, s: (b, s, 0)),
        out_shape=jax.ShapeDtypeStruct((B, s_dim, cout), jnp.float32),
    )(g, cpad, w1, s1, b1, w2, s2, b2, w3, s3, b3)


def _layer_params(params, si):
    inv = np.float32(1.0 / np.sqrt(1.0 + BN_EPS))
    out = []
    for li in range(3):
        w = params['sa%d_w%d' % (si, li)]
        din = MLP_DIMS[si][li] if li else _pad16(MLP_DIMS[si][0])
        wt = jnp.zeros((din, w.shape[0]), jnp.float32)
        wt = wt.at[:w.shape[1], :].set(w.T)
        s = (inv * params['sa%d_g%d' % (si, li)])[None, :]
        b = params['sa%d_b%d' % (si, li)][None, :]
        out.append((wt, s, b))
    return out


def _pad16(c):
    return ((c + 15) // 16) * 16


def _pad_cols(x, d):
    return jnp.pad(x, ((0, 0),) * (x.ndim - 1) + ((0, d - x.shape[-1]),))


def kernel(pointcloud, params):
    xyz = pointcloud[..., 0:3]
    x, y, z = xyz[..., 0], xyz[..., 1], xyz[..., 2]

    cx0, cy0, cz0, cx1, cy1, cz1, cx2, cy2, cz2 = _fps_all(x, y, z)
    nxyz0 = jnp.stack([cx0, cy0, cz0], axis=-1)
    nxyz1 = jnp.stack([cx1, cy1, cz1], axis=-1)
    nxyz2 = jnp.stack([cx2, cy2, cz2], axis=-1)

    s0, k0 = NPOINTS[0], NSAMPLES[0]
    xt8 = jnp.pad(jnp.stack([x, y, z], axis=1), ((0, 0), (0, 5), (0, 0)))
    gidx0 = _ball_query(_pad_cols(nxyz0, 8), xt8, RADII[0], k0)
    flat0 = (gidx0 + (jnp.arange(B, dtype=jnp.int32) * N)[:, None, None])
    table0 = _pad_cols(xyz.reshape(B * N, 3), 16)
    g0 = _gather_rows(table0, flat0.reshape(-1))
    f1 = _mlp_max(g0, _pad_cols(nxyz0, 16), _layer_params(params, 0),
                  ks=k0, sb=8)

    s1_, k1 = NPOINTS[1], NSAMPLES[1]
    x1t8 = jnp.pad(jnp.stack([cx0, cy0, cz0], axis=1), ((0, 0), (0, 5), (0, 0)))
    gidx1 = _ball_query(_pad_cols(nxyz1, 8), x1t8, RADII[1], k1)
    flat1 = (gidx1 + (jnp.arange(B, dtype=jnp.int32) * s0)[:, None, None])
    table1 = _pad_cols(jnp.concatenate([nxyz0, f1], axis=-1).reshape(B * s0, 67), 80)
    g1 = _gather_rows(table1, flat1.reshape(-1))
    f2 = _mlp_max(g1, _pad_cols(nxyz1, 80), _layer_params(params, 1),
                  ks=k1, sb=16)

    s2_, k2 = NPOINTS[2], NSAMPLES[2]
    x2t8 = jnp.pad(jnp.stack([cx1, cy1, cz1], axis=1), ((0, 0), (0, 5), (0, 0)))
    gidx2 = _ball_query(_pad_cols(nxyz2, 8), x2t8, RADII[2], k2)
    flat2 = (gidx2 + (jnp.arange(B, dtype=jnp.int32) * s1_)[:, None, None])
    table2 = _pad_cols(jnp.concatenate([nxyz1, f2], axis=-1).reshape(B * s1_, 259), 272)
    g2 = _gather_rows(table2, flat2.reshape(-1))
    f3 = _mlp_max(g2, _pad_cols(nxyz2, 272), _layer_params(params, 2),
                  ks=k2, sb=32)

    g3 = _pad_cols(jnp.concatenate([nxyz2, f3], axis=-1).reshape(B * 32, 515), 528)
    czero = jnp.zeros((B, 1, 528), jnp.float32)
    f4 = _mlp_max(g3, czero, _layer_params(params, 3), ks=32, sb=1)
    return f4.reshape(B, 1024)


# speedup vs baseline: 22.8665x; 3.3630x over previous
import functools

import jax
import jax.numpy as jnp
import numpy as np
from jax import lax
from jax.experimental import pallas as pl
from jax.experimental.pallas import tpu as pltpu
from jax.experimental.pallas import tpu_sc as plsc

B, N = 8, 8192
NPOINTS = [512, 128, 32]
RADII = [0.04, 0.08, 0.16]
NSAMPLES = [64, 32, 16]
BN_EPS = 1e-5
MLP_DIMS = [[3, 32, 32, 64], [67, 128, 128, 256], [259, 256, 512, 512],
            [515, 512, 1024, 1024]]

_NW = 32


def _fps_stage(xs, ys, zs, npoint, cx_ref, cy_ref, cz_ref, dists_ref):
    Bb, Nn = xs.shape
    iota = lax.broadcasted_iota(jnp.int32, (Bb, Nn), 1)
    iota_s = lax.broadcasted_iota(jnp.int32, (Bb, npoint), 1)
    cx_ref[...] = jnp.zeros((Bb, npoint), jnp.float32)
    cy_ref[...] = jnp.zeros((Bb, npoint), jnp.float32)
    cz_ref[...] = jnp.zeros((Bb, npoint), jnp.float32)
    dists_ref[...] = jnp.full((Bb, Nn), 1e10, jnp.float32)

    def body(i, far):
        sel = iota == far
        cx = jnp.sum(jnp.where(sel, xs, 0.0), axis=1, keepdims=True)
        cy = jnp.sum(jnp.where(sel, ys, 0.0), axis=1, keepdims=True)
        cz = jnp.sum(jnp.where(sel, zs, 0.0), axis=1, keepdims=True)
        hot = iota_s == i
        cx_ref[...] = jnp.where(hot, cx, cx_ref[...])
        cy_ref[...] = jnp.where(hot, cy, cy_ref[...])
        cz_ref[...] = jnp.where(hot, cz, cz_ref[...])
        d = (xs - cx) ** 2 + (ys - cy) ** 2 + (zs - cz) ** 2
        dists = jnp.minimum(dists_ref[...], d)
        dists_ref[...] = dists
        m = jnp.max(dists, axis=1, keepdims=True)
        far = jnp.min(jnp.where(dists == m, iota, Nn), axis=1, keepdims=True)
        return far.astype(jnp.int32)

    lax.fori_loop(0, npoint, body, jnp.zeros((Bb, 1), jnp.int32))


def _fps_all_kernel(x_ref, y_ref, z_ref,
                    cx0, cy0, cz0, cx1, cy1, cz1, cx2, cy2, cz2,
                    d0, d1, d2):
    _fps_stage(x_ref[...], y_ref[...], z_ref[...], NPOINTS[0],
               cx0, cy0, cz0, d0)
    _fps_stage(cx0[...], cy0[...], cz0[...], NPOINTS[1], cx1, cy1, cz1, d1)
    _fps_stage(cx1[...], cy1[...], cz1[...], NPOINTS[2], cx2, cy2, cz2, d2)


def _fps_all(x, y, z):
    outs = [jax.ShapeDtypeStruct((B, s), jnp.float32)
            for s in NPOINTS for _ in range(3)]
    return pl.pallas_call(
        _fps_all_kernel,
        out_shape=tuple(outs),
        scratch_shapes=[pltpu.VMEM((B, N), jnp.float32),
                        pltpu.VMEM((B, NPOINTS[0]), jnp.float32),
                        pltpu.VMEM((B, NPOINTS[1]), jnp.float32)],
    )(x, y, z)


def _ballq_kernel(nxyz_ref, xt_ref, out_ref, *, radius, nsample):
    a = nxyz_ref[0]
    bt = xt_ref[0]
    S = a.shape[0]
    Nn = bt.shape[1]
    aa = jnp.sum(a[:, 0:3] * a[:, 0:3], axis=1, keepdims=True)
    bb = jnp.sum(bt[0:3, :] * bt[0:3, :], axis=0, keepdims=True)
    ab = jnp.dot(a, bt, preferred_element_type=jnp.float32)
    sqr = aa + bb - 2.0 * ab
    iota = lax.broadcasted_iota(jnp.int32, (S, Nn), 1)
    val0 = jnp.where(sqr > radius * radius, Nn, iota)
    iota_k = lax.broadcasted_iota(jnp.int32, (S, nsample), 1)

    def body(k, state):
        val, acc = state
        m = jnp.min(val, axis=1, keepdims=True)
        acc = jnp.where(iota_k == k, m, acc)
        val = jnp.where(val == m, Nn, val)
        return val, acc

    _, idx = lax.fori_loop(0, nsample, body,
                           (val0, jnp.zeros((S, nsample), jnp.int32)))
    first = idx[:, 0:1]
    first = jnp.where(first == Nn, 0, first)
    idx = jnp.where(idx == Nn, first, idx)
    out_ref[0] = idx


def _ball_query(new_xyz8, xt8, radius, nsample):
    _, S, _ = new_xyz8.shape
    Nn = xt8.shape[2]
    return pl.pallas_call(
        functools.partial(_ballq_kernel, radius=radius, nsample=nsample),
        grid=(B,),
        in_specs=[pl.BlockSpec((1, S, 8), lambda b: (b, 0, 0)),
                  pl.BlockSpec((1, 8, Nn), lambda b: (b, 0, 0))],
        out_specs=pl.BlockSpec((1, S, nsample), lambda b: (b, 0, 0)),
        out_shape=jax.ShapeDtypeStruct((B, S, nsample), jnp.int32),
    )(new_xyz8, xt8)


def _gather_rows(table, idx):
    rows, d = idx.shape[0], table.shape[1]
    rows_pw = rows // _NW
    nchunks = rows_pw // 128
    idx3 = idx.reshape(_NW, nchunks, 128)
    mesh = plsc.VectorSubcoreMesh(core_axis_name="c", subcore_axis_name="s")

    @functools.partial(
        pl.kernel, mesh=mesh,
        compiler_params=pltpu.CompilerParams(use_tc_tiling_on_sc=False),
        out_type=jax.ShapeDtypeStruct((rows, d), jnp.float32),
        scratch_types=[pltpu.VMEM((nchunks, 128), jnp.int32),
                       pltpu.VMEM((128, d), jnp.float32),
                       pltpu.SemaphoreType.DMA],
    )
    def k(table_hbm, idx_hbm, out_hbm, idx_v, rows_v, sem):
        wid = lax.axis_index("s") * 2 + lax.axis_index("c")
        pltpu.sync_copy(idx_hbm.at[wid], idx_v)

        def body(j, carry):
            pltpu.async_copy(table_hbm.at[idx_v.at[j]], rows_v, sem).wait()
            pltpu.sync_copy(
                rows_v, out_hbm.at[pl.ds(wid * rows_pw + j * 128, 128)])
            return carry

        lax.fori_loop(0, nchunks, body, 0)

    return k(table, idx3)


S0K, K0K = 512, 64
NWORDS = N // 16


def _pack_kernel(nxyz_ref, xt_ref, out_ref, *, radius):
    a = nxyz_ref[0]
    bt = xt_ref[0]
    aa = jnp.sum(a[:, 0:3] * a[:, 0:3], axis=1, keepdims=True)
    bb = jnp.sum(bt[0:3, :] * bt[0:3, :], axis=0, keepdims=True)
    ab = jnp.dot(a, bt, preferred_element_type=jnp.float32)
    sqr = aa + bb - 2.0 * ab
    maskf = jnp.where(sqr > radius * radius, 0.0, 1.0)
    ji = lax.broadcasted_iota(jnp.int32, (512, 32), 0)
    wi = lax.broadcasted_iota(jnp.int32, (512, 32), 1)
    pmat = jnp.where((ji >> 4) == wi, 1 << (ji & 15), 0).astype(jnp.float32)
    cols = [jnp.dot(maskf[:, nb * 512:(nb + 1) * 512], pmat,
                    preferred_element_type=jnp.float32)
            for nb in range(N // 512)]
    out_ref[0] = jnp.concatenate(cols, axis=1).astype(jnp.int32)


def _pack_words(nxyz8, xt8, radius):
    return pl.pallas_call(
        functools.partial(_pack_kernel, radius=radius),
        grid=(B,),
        in_specs=[pl.BlockSpec((1, S0K, 8), lambda b: (b, 0, 0)),
                  pl.BlockSpec((1, 8, N), lambda b: (b, 0, 0))],
        out_specs=pl.BlockSpec((1, S0K, NWORDS), lambda b: (b, 0, 0)),
        out_shape=jax.ShapeDtypeStruct((B, S0K, NWORDS), jnp.int32),
    )(nxyz8, xt8)


def _sc_ballq_gather0(words, x, y, z):
    mesh = plsc.VectorSubcoreMesh(core_axis_name="c", subcore_axis_name="s")
    rows_per_sub = S0K // 4

    @functools.partial(
        pl.kernel, mesh=mesh,
        compiler_params=pltpu.CompilerParams(use_tc_tiling_on_sc=False,
                                             needs_layout_passes=False),
        out_type=jax.ShapeDtypeStruct((B * S0K * K0K, 16), jnp.float32),
        scratch_types=[pltpu.VMEM((N,), jnp.float32),
                       pltpu.VMEM((N,), jnp.float32),
                       pltpu.VMEM((N,), jnp.float32),
                       pltpu.VMEM((rows_per_sub, NWORDS), jnp.int32),
                       pltpu.VMEM((K0K,), jnp.int32),
                       pltpu.VMEM((8 * K0K, 16), jnp.float32)],
    )
    def k(words_hbm, x_hbm, y_hbm, z_hbm, out_hbm, xb, yb, zb, wb, idxb, grow):
        wid = lax.axis_index("s") * 2 + lax.axis_index("c")
        b = wid // 4
        r0 = (wid % 4) * rows_per_sub
        pltpu.sync_copy(x_hbm.at[b], xb)
        pltpu.sync_copy(y_hbm.at[b], yb)
        pltpu.sync_copy(z_hbm.at[b], zb)
        pltpu.sync_copy(words_hbm.at[b, pl.ds(r0, rows_per_sub)], wb)
        lane = lax.broadcasted_iota(jnp.int32, (16,), 0)
        zf = jnp.zeros((16,), jnp.float32)

        def zbody(i, c):
            grow[i] = zf
            return c
        lax.fori_loop(0, 8 * K0K, zbody, 0)

        def row_body(r, carry):
            def tbody(t, cnt_v):
                v = wb[r, pl.ds(t * 16, 16)]
                nz = jnp.max(v, axis=0)

                def decode(cnt_v):
                    for j in range(16):
                        w_s = v[j]
                        m = ((w_s >> lane) & 1) != 0
                        pos = cnt_v + plsc.cumsum(m.astype(jnp.int32)) - 1
                        mc = m & (pos < K0K)
                        ids = (t * 16 + j) * 16 + lane
                        plsc.store_scatter(idxb, [pos], ids, mask=mc)
                        cnt_v = cnt_v + plsc.all_reduce_population_count(mc)
                    return cnt_v

                return lax.cond(nz > 0, decode, lambda c: c, cnt_v)

            cnt_v = lax.fori_loop(0, NWORDS // 16, tbody,
                                  jnp.zeros((16,), jnp.int32))
            v0 = idxb[pl.ds(0, 16)]
            first_s = jnp.where(cnt_v[0] > 0, v0[0], 0)
            rr = lax.bitwise_and(r, 7)
            for t2 in range(K0K // 16):
                lane_ids = t2 * 16 + lane
                cur = idxb[pl.ds(t2 * 16, 16)]
                idxv = jnp.where(lane_ids < cnt_v, cur, first_s)
                gx = plsc.load_gather(xb, [idxv])
                gy = plsc.load_gather(yb, [idxv])
                gz = plsc.load_gather(zb, [idxv])
                rowpos = rr * K0K + t2 * 16 + lane
                plsc.store_scatter(grow, [rowpos, jnp.zeros((16,), jnp.int32)], gx)
                plsc.store_scatter(grow, [rowpos, jnp.full((16,), 1, jnp.int32)], gy)
                plsc.store_scatter(grow, [rowpos, jnp.full((16,), 2, jnp.int32)], gz)

            @pl.when(rr == 7)
            def _():
                base = (b * S0K + r0 + r - 7) * K0K
                pltpu.sync_copy(grow, out_hbm.at[pl.ds(base, 8 * K0K)])

            return carry

        lax.fori_loop(0, rows_per_sub, row_body, 0)

    return k(words, x, y, z)


def _mlp_kernel(g_ref, c_ref, w1, s1, b1, w2, s2, b2, w3, s3, b3, out_ref,
                *, sb, ks):
    d = g_ref.shape[1]
    x = g_ref[...].reshape(sb, ks, d) - c_ref[0][:, None, :]
    x = x.reshape(sb * ks, d)
    for w, s, b in ((w1, s1, b1), (w2, s2, b2), (w3, s3, b3)):
        x = jnp.dot(x, w[...], preferred_element_type=jnp.float32)
        x = jnp.maximum(x * s[...] + b[...], 0.0)
    cout = x.shape[1]
    out_ref[0] = jnp.max(x.reshape(sb, ks, cout), axis=1)


def _mlp_max(g, cpad, weights, ks, sb):
    _, s_dim, d = cpad.shape
    (w1, s1, b1), (w2, s2, b2), (w3, s3, b3) = weights
    cout = w3.shape[1]
    n_sb = s_dim // sb
    wspec = [pl.BlockSpec(w.shape, lambda b, s: tuple([0] * w.ndim))
             for w in (w1, s1, b1, w2, s2, b2, w3, s3, b3)]
    return pl.pallas_call(
        functools.partial(_mlp_kernel, sb=sb, ks=ks),
        grid=(B, n_sb),
        in_specs=[pl.BlockSpec((sb * ks, d),
                               lambda b, s, n_sb=n_sb: (b * n_sb + s, 0)),
                  pl.BlockSpec((1, sb, d), lambda b, s: (b, s, 0))] + wspec,
        out_specs=pl.BlockSpec((1, sb, cout), lambda b, s: (b, s, 0)),
        out_shape=jax.ShapeDtypeStruct((B, s_dim, cout), jnp.float32),
    )(g, cpad, w1, s1, b1, w2, s2, b2, w3, s3, b3)


def _layer_params(params, si):
    inv = np.float32(1.0 / np.sqrt(1.0 + BN_EPS))
    out = []
    for li in range(3):
        w = params['sa%d_w%d' % (si, li)]
        din = MLP_DIMS[si][li] if li else _pad16(MLP_DIMS[si][0])
        wt = jnp.zeros((din, w.shape[0]), jnp.float32)
        wt = wt.at[:w.shape[1], :].set(w.T)
        s = (inv * params['sa%d_g%d' % (si, li)])[None, :]
        b = params['sa%d_b%d' % (si, li)][None, :]
        out.append((wt, s, b))
    return out


def _pad16(c):
    return ((c + 15) // 16) * 16


def _pad_cols(x, d):
    return jnp.pad(x, ((0, 0),) * (x.ndim - 1) + ((0, d - x.shape[-1]),))


def kernel(pointcloud, params):
    xyz = pointcloud[..., 0:3]
    x, y, z = xyz[..., 0], xyz[..., 1], xyz[..., 2]

    cx0, cy0, cz0, cx1, cy1, cz1, cx2, cy2, cz2 = _fps_all(x, y, z)
    nxyz0 = jnp.stack([cx0, cy0, cz0], axis=-1)
    nxyz1 = jnp.stack([cx1, cy1, cz1], axis=-1)
    nxyz2 = jnp.stack([cx2, cy2, cz2], axis=-1)

    s0, k0 = NPOINTS[0], NSAMPLES[0]
    xt8 = jnp.pad(jnp.stack([x, y, z], axis=1), ((0, 0), (0, 5), (0, 0)))
    words0 = _pack_words(_pad_cols(nxyz0, 8), xt8, RADII[0])
    g0 = _sc_ballq_gather0(words0, x, y, z)
    f1 = _mlp_max(g0, _pad_cols(nxyz0, 16), _layer_params(params, 0),
                  ks=k0, sb=8)

    s1_, k1 = NPOINTS[1], NSAMPLES[1]
    x1t8 = jnp.pad(jnp.stack([cx0, cy0, cz0], axis=1), ((0, 0), (0, 5), (0, 0)))
    gidx1 = _ball_query(_pad_cols(nxyz1, 8), x1t8, RADII[1], k1)
    flat1 = (gidx1 + (jnp.arange(B, dtype=jnp.int32) * s0)[:, None, None])
    table1 = _pad_cols(jnp.concatenate([nxyz0, f1], axis=-1).reshape(B * s0, 67), 80)
    g1 = _gather_rows(table1, flat1.reshape(-1))
    f2 = _mlp_max(g1, _pad_cols(nxyz1, 80), _layer_params(params, 1),
                  ks=k1, sb=16)

    s2_, k2 = NPOINTS[2], NSAMPLES[2]
    x2t8 = jnp.pad(jnp.stack([cx1, cy1, cz1], axis=1), ((0, 0), (0, 5), (0, 0)))
    gidx2 = _ball_query(_pad_cols(nxyz2, 8), x2t8, RADII[2], k2)
    flat2 = (gidx2 + (jnp.arange(B, dtype=jnp.int32) * s1_)[:, None, None])
    table2 = _pad_cols(jnp.concatenate([nxyz1, f2], axis=-1).reshape(B * s1_, 259), 272)
    g2 = _gather_rows(table2, flat2.reshape(-1))
    f3 = _mlp_max(g2, _pad_cols(nxyz2, 272), _layer_params(params, 2),
                  ks=k2, sb=32)

    g3 = _pad_cols(jnp.concatenate([nxyz2, f3], axis=-1).reshape(B * 32, 515), 528)
    czero = jnp.zeros((B, 1, 528), jnp.float32)
    f4 = _mlp_max(g3, czero, _layer_params(params, 3), ks=32, sb=1)
    return f4.reshape(B, 1024)
